# Initial kernel scaffold; baseline (speedup 1.0000x reference)
import jax
import jax.numpy as jnp
from jax.experimental import pallas as pl


def kernel(pred_logits, pred_boxes, orig_target_sizes):
    raise NotImplementedError("write your pallas kernel here")


# speedup vs baseline: 5.2204x; 5.2204x over previous
import functools

import jax
import jax.numpy as jnp
from jax import lax
from jax.experimental import pallas as pl
from jax.experimental.pallas import tpu as pltpu

IOU_THRESHOLD = 0.5
SCORE_THRESHOLD = 0.6
KEEP_TOPK = 20

B = 8
N = 20000
C = 80
R = 1024
NB = 20
NSUB = NB * 8

NEG = float("-inf")
BIGI = 2**30


def _nms_kernel(sizes_ref, lg_ref, bx_ref, lab_out, box_out, sc_out,
                sc_s, lab_s, co_s, mc_s):
    b = pl.program_id(0)
    j = pl.program_id(1)

    x = lg_ref[0]
    m = jnp.max(x, axis=1)
    ci = lax.broadcasted_iota(jnp.int32, (R, C), 1)
    lb = jnp.min(jnp.where(x == m[:, None], ci, BIGI), axis=1)
    p = jax.nn.sigmoid(m)
    col = j * R + lax.broadcasted_iota(jnp.int32, (R,), 0)
    valid = col < N
    sm = jnp.where(valid & (p > SCORE_THRESHOLD), p, NEG)
    sc_s[b, j] = sm.reshape(8, 128)
    lab_s[b, j] = jnp.where(valid, lb, 0).reshape(8, 128)

    bb = bx_ref[0]
    cxy = bb[0:2]
    wh = bb[2:4]
    xy = jnp.concatenate([cxy - 0.5 * wh, cxy + 0.5 * wh], axis=0)
    rio = lax.broadcasted_iota(jnp.int32, (4, R), 0)
    wsf = sizes_ref[b, 0].astype(jnp.float32)
    hsf = sizes_ref[b, 1].astype(jnp.float32)
    xy = xy * jnp.where(rio % 2 == 0, wsf, hsf)
    vcol = (j * R + lax.broadcasted_iota(jnp.int32, (4, R), 1)) < N
    xy = jnp.where(vcol, xy, 0.0)
    co_s[b, :, j] = xy.reshape(4, 8, 128)

    am = jnp.abs(xy)

    @pl.when(j == 0)
    def _():
        mc_s[b] = am

    @pl.when(j > 0)
    def _():
        mc_s[b] = jnp.maximum(mc_s[b], am)

    @pl.when((b == B - 1) & (j == NB - 1))
    def _nms():
        S0 = sc_s[...].reshape(B, NSUB, 128)
        L = lab_s[...].reshape(B, NSUB, 128)
        Cc = co_s[...]
        X1 = Cc[:, 0].reshape(B, NSUB, 128)
        Y1 = Cc[:, 1].reshape(B, NSUB, 128)
        X2 = Cc[:, 2].reshape(B, NSUB, 128)
        Y2 = Cc[:, 3].reshape(B, NSUB, 128)
        mc = jnp.max(mc_s[...], axis=(1, 2)) + 1.0
        off = L.astype(jnp.float32) * mc[:, None, None]
        Xo1 = X1 + off
        Yo1 = Y1 + off
        Xo2 = X2 + off
        Yo2 = Y2 + off
        area_b = (Xo2 - Xo1) * (Yo2 - Yo1)
        flat = (lax.broadcasted_iota(jnp.int32, (B, NSUB, 128), 1) * 128
                + lax.broadcasted_iota(jnp.int32, (B, NSUB, 128), 2))
        kcol = lax.broadcasted_iota(jnp.int32, (B, KEEP_TOPK), 1)

        def pick(selm, A, fill):
            return jnp.max(jnp.where(selm, A, fill), axis=(1, 2))

        def body(k, carry):
            S, sc_a, lb_a, b1, b2, b3, b4 = carry
            m = jnp.max(S, axis=(1, 2))
            eq = S == m[:, None, None]
            idx = jnp.min(jnp.where(eq, flat, BIGI), axis=(1, 2))
            selm = flat == idx[:, None, None]
            xo1 = pick(selm, Xo1, NEG)
            yo1 = pick(selm, Yo1, NEG)
            xo2 = pick(selm, Xo2, NEG)
            yo2 = pick(selm, Yo2, NEG)
            lab = pick(selm, L, -1)
            xx1 = jnp.maximum(Xo1, xo1[:, None, None])
            yy1 = jnp.maximum(Yo1, yo1[:, None, None])
            xx2 = jnp.minimum(Xo2, xo2[:, None, None])
            yy2 = jnp.minimum(Yo2, yo2[:, None, None])
            inter = (jnp.maximum(xx2 - xx1, 0.0)
                     * jnp.maximum(yy2 - yy1, 0.0))
            area_a = (xo2 - xo1) * (yo2 - yo1)
            iou = inter / (area_a[:, None, None] + area_b - inter + 1e-9)
            S = jnp.where(iou > IOU_THRESHOLD, NEG, S)
            S = jnp.where(selm, NEG, S)
            t = lab.astype(jnp.float32) * mc
            sel = kcol == k
            sc_a = jnp.where(sel, m[:, None], sc_a)
            lb_a = jnp.where(sel, lab[:, None], lb_a)
            b1 = jnp.where(sel, (xo1 - t)[:, None], b1)
            b2 = jnp.where(sel, (yo1 - t)[:, None], b2)
            b3 = jnp.where(sel, (xo2 - t)[:, None], b3)
            b4 = jnp.where(sel, (yo2 - t)[:, None], b4)
            return S, sc_a, lb_a, b1, b2, b3, b4

        z = jnp.zeros((B, KEEP_TOPK), jnp.float32)
        zi = jnp.zeros((B, KEEP_TOPK), jnp.int32)
        _, sc_a, lb_a, b1, b2, b3, b4 = lax.fori_loop(
            0, KEEP_TOPK, body, (S0, z, zi, z, z, z, z))
        sc_out[...] = sc_a
        lab_out[...] = lb_a
        box_out[...] = jnp.stack([b1, b2, b3, b4], axis=-1)


@jax.jit
def kernel(pred_logits, pred_boxes, orig_target_sizes):
    bx_t = jnp.transpose(pred_boxes, (0, 2, 1))
    grid = (B, NB)
    out = pl.pallas_call(
        _nms_kernel,
        grid=grid,
        in_specs=[
            pl.BlockSpec(memory_space=pltpu.SMEM),
            pl.BlockSpec((1, R, C), lambda b, j: (b, j, 0)),
            pl.BlockSpec((1, 4, R), lambda b, j: (b, 0, j)),
        ],
        out_specs=[
            pl.BlockSpec((B, KEEP_TOPK), lambda b, j: (0, 0)),
            pl.BlockSpec((B, KEEP_TOPK, 4), lambda b, j: (0, 0, 0)),
            pl.BlockSpec((B, KEEP_TOPK), lambda b, j: (0, 0)),
        ],
        out_shape=[
            jax.ShapeDtypeStruct((B, KEEP_TOPK), jnp.int32),
            jax.ShapeDtypeStruct((B, KEEP_TOPK, 4), jnp.float32),
            jax.ShapeDtypeStruct((B, KEEP_TOPK), jnp.float32),
        ],
        scratch_shapes=[
            pltpu.VMEM((B, NB, 8, 128), jnp.float32),
            pltpu.VMEM((B, NB, 8, 128), jnp.int32),
            pltpu.VMEM((B, 4, NB, 8, 128), jnp.float32),
            pltpu.VMEM((B, 4, R), jnp.float32),
        ],
    )(orig_target_sizes, pred_logits, bx_t)
    return out[0], out[1], out[2]


# speedup vs baseline: 18.2912x; 3.5038x over previous
import functools

import jax
import jax.numpy as jnp
from jax import lax
from jax.experimental import pallas as pl
from jax.experimental.pallas import tpu as pltpu

IOU_THRESHOLD = 0.5
SCORE_THRESHOLD = 0.6
KEEP_TOPK = 20

B = 8
N = 20000
C = 80
OCT = C // 8
R = 4096
NB = 5
RS = R // 128
NSUB = NB * RS

NEG = float("-inf")
BIGI = 2**30


def _nms_kernel(sizes_ref, lg_ref, bx_ref, lab_out, box_out, sc_out,
                sc_s, lab_s, co_s, mc_s):
    b = pl.program_id(0)
    j = pl.program_id(1)

    x = lg_ref[0].reshape(OCT, 8, R)
    m2 = jnp.max(x, axis=0)
    mf = m2
    for k in (4, 2, 1):
        mf = jnp.maximum(mf, pltpu.roll(mf, k, 0))
    ci = (lax.broadcasted_iota(jnp.int32, (OCT, 8, R), 0) * 8
          + lax.broadcasted_iota(jnp.int32, (OCT, 8, R), 1))
    lb2 = jnp.min(jnp.where(x == mf[None], ci, BIGI), axis=0)
    lf = lb2
    for k in (4, 2, 1):
        lf = jnp.minimum(lf, pltpu.roll(lf, k, 0))
    p = jax.nn.sigmoid(mf[0])
    col = j * R + lax.broadcasted_iota(jnp.int32, (R,), 0)
    valid = col < N
    sm = jnp.where(valid & (p > SCORE_THRESHOLD), p, NEG)
    sc_s[b, j] = sm.reshape(RS, 128)
    lab_s[b, j] = jnp.where(valid, lf[0], 0).reshape(RS, 128)

    bb = bx_ref[0]
    cxy = bb[0:2]
    wh = bb[2:4]
    xy = jnp.concatenate([cxy - 0.5 * wh, cxy + 0.5 * wh], axis=0)
    rio = lax.broadcasted_iota(jnp.int32, (4, R), 0)
    wsf = sizes_ref[b, 0].astype(jnp.float32)
    hsf = sizes_ref[b, 1].astype(jnp.float32)
    xy = xy * jnp.where(rio % 2 == 0, wsf, hsf)
    vcol = (j * R + lax.broadcasted_iota(jnp.int32, (4, R), 1)) < N
    xy = jnp.where(vcol, xy, 0.0)
    co_s[b, :, j] = xy.reshape(4, RS, 128)

    am = jnp.abs(xy)

    @pl.when(j == 0)
    def _():
        mc_s[b] = am

    @pl.when(j > 0)
    def _():
        mc_s[b] = jnp.maximum(mc_s[b], am)

    @pl.when((b == B - 1) & (j == NB - 1))
    def _nms():
        S0 = sc_s[...].reshape(B, NSUB, 128)
        L = lab_s[...].reshape(B, NSUB, 128)
        Cc = co_s[...]
        X1 = Cc[:, 0].reshape(B, NSUB, 128)
        Y1 = Cc[:, 1].reshape(B, NSUB, 128)
        X2 = Cc[:, 2].reshape(B, NSUB, 128)
        Y2 = Cc[:, 3].reshape(B, NSUB, 128)
        mc = jnp.max(mc_s[...], axis=(1, 2)) + 1.0
        off = L.astype(jnp.float32) * mc[:, None, None]
        Xo1 = X1 + off
        Yo1 = Y1 + off
        Xo2 = X2 + off
        Yo2 = Y2 + off
        area_b = (Xo2 - Xo1) * (Yo2 - Yo1)
        flat = (lax.broadcasted_iota(jnp.int32, (B, NSUB, 128), 1) * 128
                + lax.broadcasted_iota(jnp.int32, (B, NSUB, 128), 2))
        kcol = lax.broadcasted_iota(jnp.int32, (B, KEEP_TOPK), 1)

        def pick(selm, A, fill):
            return jnp.max(jnp.where(selm, A, fill), axis=(1, 2))

        def body(k, carry):
            S, sc_a, lb_a, b1, b2, b3, b4 = carry
            m = jnp.max(S, axis=(1, 2))
            eq = S == m[:, None, None]
            idx = jnp.min(jnp.where(eq, flat, BIGI), axis=(1, 2))
            selm = flat == idx[:, None, None]
            xo1 = pick(selm, Xo1, NEG)
            yo1 = pick(selm, Yo1, NEG)
            xo2 = pick(selm, Xo2, NEG)
            yo2 = pick(selm, Yo2, NEG)
            lab = pick(selm, L, -1)
            xx1 = jnp.maximum(Xo1, xo1[:, None, None])
            yy1 = jnp.maximum(Yo1, yo1[:, None, None])
            xx2 = jnp.minimum(Xo2, xo2[:, None, None])
            yy2 = jnp.minimum(Yo2, yo2[:, None, None])
            inter = (jnp.maximum(xx2 - xx1, 0.0)
                     * jnp.maximum(yy2 - yy1, 0.0))
            area_a = (xo2 - xo1) * (yo2 - yo1)
            iou = inter / (area_a[:, None, None] + area_b - inter + 1e-9)
            S = jnp.where(iou > IOU_THRESHOLD, NEG, S)
            S = jnp.where(selm, NEG, S)
            t = lab.astype(jnp.float32) * mc
            sel = kcol == k
            sc_a = jnp.where(sel, m[:, None], sc_a)
            lb_a = jnp.where(sel, lab[:, None], lb_a)
            b1 = jnp.where(sel, (xo1 - t)[:, None], b1)
            b2 = jnp.where(sel, (yo1 - t)[:, None], b2)
            b3 = jnp.where(sel, (xo2 - t)[:, None], b3)
            b4 = jnp.where(sel, (yo2 - t)[:, None], b4)
            return S, sc_a, lb_a, b1, b2, b3, b4

        z = jnp.zeros((B, KEEP_TOPK), jnp.float32)
        zi = jnp.zeros((B, KEEP_TOPK), jnp.int32)
        _, sc_a, lb_a, b1, b2, b3, b4 = lax.fori_loop(
            0, KEEP_TOPK, body, (S0, z, zi, z, z, z, z))
        sc_out[...] = sc_a
        lab_out[...] = lb_a
        box_out[...] = jnp.stack([b1, b2, b3, b4], axis=-1)


@jax.jit
def kernel(pred_logits, pred_boxes, orig_target_sizes):
    bx_t = jnp.transpose(pred_boxes, (0, 2, 1))
    lg_t = jnp.transpose(pred_logits, (0, 2, 1))
    grid = (B, NB)
    out = pl.pallas_call(
        _nms_kernel,
        grid=grid,
        in_specs=[
            pl.BlockSpec(memory_space=pltpu.SMEM),
            pl.BlockSpec((1, C, R), lambda b, j: (b, 0, j)),
            pl.BlockSpec((1, 4, R), lambda b, j: (b, 0, j)),
        ],
        out_specs=[
            pl.BlockSpec((B, KEEP_TOPK), lambda b, j: (0, 0)),
            pl.BlockSpec((B, KEEP_TOPK, 4), lambda b, j: (0, 0, 0)),
            pl.BlockSpec((B, KEEP_TOPK), lambda b, j: (0, 0)),
        ],
        out_shape=[
            jax.ShapeDtypeStruct((B, KEEP_TOPK), jnp.int32),
            jax.ShapeDtypeStruct((B, KEEP_TOPK, 4), jnp.float32),
            jax.ShapeDtypeStruct((B, KEEP_TOPK), jnp.float32),
        ],
        scratch_shapes=[
            pltpu.VMEM((B, NB, RS, 128), jnp.float32),
            pltpu.VMEM((B, NB, RS, 128), jnp.int32),
            pltpu.VMEM((B, 4, NB, RS, 128), jnp.float32),
            pltpu.VMEM((B, 4, R), jnp.float32),
        ],
    )(orig_target_sizes, lg_t, bx_t)
    return out[0], out[1], out[2]
